# baseline (speedup 1.0000x reference)
import functools

import jax
import jax.numpy as jnp
from jax import lax
from jax.experimental import pallas as pl
from jax.experimental.pallas import tpu as pltpu
from jax.experimental.pallas import tpu_sc as plsc

N = 10000
E = 320000
D = 128
DH = 64
DW = 80
NPAD = 10240
K = 128
NCHUNK = 160
EPT = NCHUNK * K
EPAD = 16 * EPT
NBUF = 2
IB = 8
NBLK = NCHUNK // IB
ROWS_PER_TILE = NPAD // 16


def _sc_segment_sum(xa2, src2, dst2):
  mesh = plsc.VectorSubcoreMesh(core_axis_name="c", subcore_axis_name="s")

  @functools.partial(
      pl.kernel,
      mesh=mesh,
      out_type=jax.ShapeDtypeStruct((2 * NPAD, DW), jnp.float32),
      scratch_types=[
          pltpu.VMEM((2, IB, K), jnp.int32),
          pltpu.VMEM((2, IB, K), jnp.int32),
          pltpu.VMEM((NBUF, K, DW), jnp.float32),
          pltpu.VMEM_SHARED((NPAD, DW), jnp.float32),
          pltpu.VMEM_SHARED((NPAD, DW), jnp.float32),
          [pltpu.SemaphoreType.DMA] * NBUF,
          [pltpu.SemaphoreType.DMA] * NBUF,
          [pltpu.SemaphoreType.DMA] * 2,
          [pltpu.SemaphoreType.DMA] * 2,
      ],
      compiler_params=pltpu.CompilerParams(use_tc_tiling_on_sc=False),
  )
  def k(xa_hbm, src_hbm, dst_hbm, out_hbm,
        src_v, dst_v, rows_v, acc_sh, xtab_sh, sem_g, sem_s, sem_is, sem_id):
    cid = lax.axis_index("c")
    sid = lax.axis_index("s")

    pltpu.sync_copy(
        xa_hbm.at[pl.ds(cid * NPAD + sid * ROWS_PER_TILE, ROWS_PER_TILE)],
        xtab_sh.at[pl.ds(sid * ROWS_PER_TILE, ROWS_PER_TILE)])

    def zrow(r, carry):
      for c in range(DW // 16):
        rows_v[jnp.int32(0), r, pl.ds(c * 16, 16)] = jnp.zeros(
            (16,), jnp.float32)
      return carry

    lax.fori_loop(jnp.int32(0), jnp.int32(K), zrow, jnp.int32(0))

    def zslab(j, carry):
      pltpu.sync_copy(rows_v.at[jnp.int32(0)],
                      acc_sh.at[pl.ds(sid * ROWS_PER_TILE + j * K, K)])
      return carry

    lax.fori_loop(jnp.int32(0), jnp.int32(ROWS_PER_TILE // K), zslab,
                  jnp.int32(0))
    plsc.subcore_barrier()

    def load_idx_block(j, p):
      pi = jnp.int32(p)
      pltpu.async_copy(src_hbm.at[sid, pl.ds(j * IB, IB)], src_v.at[pi],
                       sem_is[p])
      pltpu.async_copy(dst_hbm.at[sid, pl.ds(j * IB, IB)], dst_v.at[pi],
                       sem_id[p])

    def wait_idx_block(p):
      pi = jnp.int32(p)
      pltpu.make_async_copy(src_hbm.at[sid, pl.ds(0, IB)], src_v.at[pi],
                            sem_is[p]).wait()
      pltpu.make_async_copy(dst_hbm.at[sid, pl.ds(0, IB)], dst_v.at[pi],
                            sem_id[p]).wait()

    load_idx_block(jnp.int32(0), 0)
    load_idx_block(jnp.int32(1), 1)
    wait_idx_block(0)

    for b in range(NBUF):
      bi = jnp.int32(b)
      pltpu.async_copy(xtab_sh.at[src_v.at[jnp.int32(0), bi]],
                       rows_v.at[bi], sem_g[b])

    scat_dummy = xa_hbm.at[pl.ds(0, K)]

    def block_pair(i, carry):
      for q in range(2):
        p, pn = q, 1 - q
        j = i * jnp.int32(2) + jnp.int32(q)
        for u in range(IB):
          b = u % NBUF
          bi = jnp.int32(b)
          c = j * IB + jnp.int32(u)
          pltpu.make_async_copy(scat_dummy, rows_v.at[bi], sem_g[b]).wait()
          pltpu.async_copy(rows_v.at[bi],
                           acc_sh.at[dst_v.at[jnp.int32(p), jnp.int32(u)]],
                           sem_s[b], add=True)
          if u == IB - 2:
            @pl.when(j + jnp.int32(1) < jnp.int32(NBLK))
            def _():
              wait_idx_block(pn)
          @pl.when(c + jnp.int32(NBUF) < jnp.int32(NCHUNK))
          def _():
            u2 = u + NBUF
            if u2 < IB:
              src_row = src_v.at[jnp.int32(p), jnp.int32(u2)]
            else:
              src_row = src_v.at[jnp.int32(pn), jnp.int32(u2 - IB)]
            pltpu.make_async_copy(scat_dummy, rows_v.at[bi], sem_s[b]).wait()
            pltpu.async_copy(xtab_sh.at[src_row], rows_v.at[bi], sem_g[b])
        @pl.when(j + jnp.int32(2) < jnp.int32(NBLK))
        def _():
          load_idx_block(j + jnp.int32(2), p)
      return carry

    lax.fori_loop(jnp.int32(0), jnp.int32(NBLK // 2), block_pair,
                  jnp.int32(0), unroll=False)

    for b in range(NBUF):
      bi = jnp.int32(b)
      pltpu.make_async_copy(scat_dummy, rows_v.at[bi], sem_s[b]).wait()

    plsc.subcore_barrier()

    pltpu.sync_copy(
        acc_sh.at[pl.ds(sid * ROWS_PER_TILE, ROWS_PER_TILE)],
        out_hbm.at[pl.ds(cid * NPAD + sid * ROWS_PER_TILE, ROWS_PER_TILE)])

  return k(xa2, src2, dst2)


def _tc_finish(acc, x, Wl_lo, Wl_hi, b_l, W_r):
  BN = 1000

  def body(a0_ref, a1_ref, x_ref, wlo_ref, whi_ref, wr_ref, b_ref, o_ref):
    lo = a0_ref[0]
    hi = a1_ref[0]
    cnt = jnp.maximum(lo[:, DH:DH + 1], 1.0)
    mean_lo = lo[:, :DH] / cnt
    mean_hi = hi[:, :DH] / cnt
    dn = (((1,), (1,)), ((), ()))
    o_ref[...] = (
        lax.dot_general(mean_lo, wlo_ref[...], dn,
                        preferred_element_type=jnp.float32)
        + lax.dot_general(mean_hi, whi_ref[...], dn,
                          preferred_element_type=jnp.float32)
        + lax.dot_general(x_ref[...], wr_ref[...], dn,
                          preferred_element_type=jnp.float32)
        + b_ref[...])

  return pl.pallas_call(
      body,
      grid=(N // BN,),
      in_specs=[
          pl.BlockSpec((1, BN, DW),
                       lambda i: (jnp.int32(0), i, jnp.int32(0))),
          pl.BlockSpec((1, BN, DW),
                       lambda i: (jnp.int32(1), i, jnp.int32(0))),
          pl.BlockSpec((BN, D), lambda i: (i, jnp.int32(0))),
          pl.BlockSpec((D, DH), lambda i: (jnp.int32(0), jnp.int32(0))),
          pl.BlockSpec((D, DH), lambda i: (jnp.int32(0), jnp.int32(0))),
          pl.BlockSpec((D, D), lambda i: (jnp.int32(0), jnp.int32(0))),
          pl.BlockSpec((1, D), lambda i: (jnp.int32(0), jnp.int32(0))),
      ],
      out_specs=pl.BlockSpec((BN, D), lambda i: (i, jnp.int32(0))),
      out_shape=jax.ShapeDtypeStruct((N, D), jnp.float32),
  )(acc, acc, x, Wl_lo, Wl_hi, W_r, b_l.reshape(1, D))


def kernel(x, edge_index, edge_attr, W_l, b_l, W_r):
  src = edge_index[0].astype(jnp.int32)
  dst = edge_index[1].astype(jnp.int32)
  xf = x.astype(jnp.float32)

  xa2 = jnp.zeros((2 * NPAD, DW), jnp.float32)
  xa2 = xa2.at[:N, :DH].set(xf[:, :DH])
  xa2 = xa2.at[:N, DH].set(1.0)
  xa2 = xa2.at[NPAD:NPAD + N, :DH].set(xf[:, DH:])

  pad = EPAD - E
  src_p = jnp.concatenate([src, jnp.zeros((pad,), jnp.int32)])
  dst_p = jnp.concatenate([dst, jnp.full((pad,), NPAD - 1, jnp.int32)])
  src3 = src_p.reshape(16, NCHUNK, K)
  dst3 = dst_p.reshape(16, NCHUNK, K)

  acc = _sc_segment_sum(xa2, src3, dst3).reshape(2, NPAD, DW)
  Wl = W_l.astype(jnp.float32)
  out = _tc_finish(acc, xf, Wl[:, :DH], Wl[:, DH:],
                   b_l.astype(jnp.float32), W_r.astype(jnp.float32))
  out_dtype = jnp.result_type(x.dtype, W_l.dtype)
  return out.astype(out_dtype)


# speedup vs baseline: 1.0054x; 1.0054x over previous
import functools

import jax
import jax.numpy as jnp
from jax import lax
from jax.experimental import pallas as pl
from jax.experimental.pallas import tpu as pltpu
from jax.experimental.pallas import tpu_sc as plsc

N = 10000
E = 320000
D = 128
DH = 64
DW = 80
NPAD = 10240
K = 64
NCHUNK = 320
EPT = NCHUNK * K
EPAD = 16 * EPT
NBUF = 4
IB = 8
NBLK = NCHUNK // IB
ROWS_PER_TILE = NPAD // 16


def _sc_segment_sum(xa2, src2, dst2):
  mesh = plsc.VectorSubcoreMesh(core_axis_name="c", subcore_axis_name="s")

  @functools.partial(
      pl.kernel,
      mesh=mesh,
      out_type=jax.ShapeDtypeStruct((2 * NPAD, DW), jnp.float32),
      scratch_types=[
          pltpu.VMEM((2, IB, K), jnp.int32),
          pltpu.VMEM((2, IB, K), jnp.int32),
          pltpu.VMEM((NBUF, K, DW), jnp.float32),
          pltpu.VMEM_SHARED((NPAD, DW), jnp.float32),
          pltpu.VMEM_SHARED((NPAD, DW), jnp.float32),
          [pltpu.SemaphoreType.DMA] * NBUF,
          [pltpu.SemaphoreType.DMA] * NBUF,
          [pltpu.SemaphoreType.DMA] * 2,
          [pltpu.SemaphoreType.DMA] * 2,
      ],
      compiler_params=pltpu.CompilerParams(use_tc_tiling_on_sc=False),
  )
  def k(xa_hbm, src_hbm, dst_hbm, out_hbm,
        src_v, dst_v, rows_v, acc_sh, xtab_sh, sem_g, sem_s, sem_is, sem_id):
    cid = lax.axis_index("c")
    sid = lax.axis_index("s")

    pltpu.sync_copy(
        xa_hbm.at[pl.ds(cid * NPAD + sid * ROWS_PER_TILE, ROWS_PER_TILE)],
        xtab_sh.at[pl.ds(sid * ROWS_PER_TILE, ROWS_PER_TILE)])

    def zrow(r, carry):
      for c in range(DW // 16):
        rows_v[jnp.int32(0), r, pl.ds(c * 16, 16)] = jnp.zeros(
            (16,), jnp.float32)
      return carry

    lax.fori_loop(jnp.int32(0), jnp.int32(K), zrow, jnp.int32(0))

    def zslab(j, carry):
      pltpu.sync_copy(rows_v.at[jnp.int32(0)],
                      acc_sh.at[pl.ds(sid * ROWS_PER_TILE + j * K, K)])
      return carry

    lax.fori_loop(jnp.int32(0), jnp.int32(ROWS_PER_TILE // K), zslab,
                  jnp.int32(0))
    plsc.subcore_barrier()

    def load_idx_block(j, p):
      pi = jnp.int32(p)
      pltpu.async_copy(src_hbm.at[sid, pl.ds(j * IB, IB)], src_v.at[pi],
                       sem_is[p])
      pltpu.async_copy(dst_hbm.at[sid, pl.ds(j * IB, IB)], dst_v.at[pi],
                       sem_id[p])

    def wait_idx_block(p):
      pi = jnp.int32(p)
      pltpu.make_async_copy(src_hbm.at[sid, pl.ds(0, IB)], src_v.at[pi],
                            sem_is[p]).wait()
      pltpu.make_async_copy(dst_hbm.at[sid, pl.ds(0, IB)], dst_v.at[pi],
                            sem_id[p]).wait()

    load_idx_block(jnp.int32(0), 0)
    load_idx_block(jnp.int32(1), 1)
    wait_idx_block(0)

    for b in range(NBUF):
      bi = jnp.int32(b)
      pltpu.async_copy(xtab_sh.at[src_v.at[jnp.int32(0), bi]],
                       rows_v.at[bi], sem_g[b])

    scat_dummy = xa_hbm.at[pl.ds(0, K)]

    def block_pair(i, carry):
      for q in range(2):
        p, pn = q, 1 - q
        j = i * jnp.int32(2) + jnp.int32(q)
        for u in range(IB):
          b = u % NBUF
          bi = jnp.int32(b)
          c = j * IB + jnp.int32(u)
          pltpu.make_async_copy(scat_dummy, rows_v.at[bi], sem_g[b]).wait()
          pltpu.async_copy(rows_v.at[bi],
                           acc_sh.at[dst_v.at[jnp.int32(p), jnp.int32(u)]],
                           sem_s[b], add=True)
          if u == IB - 2:
            @pl.when(j + jnp.int32(1) < jnp.int32(NBLK))
            def _():
              wait_idx_block(pn)
          @pl.when(c + jnp.int32(NBUF) < jnp.int32(NCHUNK))
          def _():
            u2 = u + NBUF
            if u2 < IB:
              src_row = src_v.at[jnp.int32(p), jnp.int32(u2)]
            else:
              src_row = src_v.at[jnp.int32(pn), jnp.int32(u2 - IB)]
            pltpu.make_async_copy(scat_dummy, rows_v.at[bi], sem_s[b]).wait()
            pltpu.async_copy(xtab_sh.at[src_row], rows_v.at[bi], sem_g[b])
        @pl.when(j + jnp.int32(2) < jnp.int32(NBLK))
        def _():
          load_idx_block(j + jnp.int32(2), p)
      return carry

    lax.fori_loop(jnp.int32(0), jnp.int32(NBLK // 2), block_pair,
                  jnp.int32(0), unroll=False)

    for b in range(NBUF):
      bi = jnp.int32(b)
      pltpu.make_async_copy(scat_dummy, rows_v.at[bi], sem_s[b]).wait()

    plsc.subcore_barrier()

    pltpu.sync_copy(
        acc_sh.at[pl.ds(sid * ROWS_PER_TILE, ROWS_PER_TILE)],
        out_hbm.at[pl.ds(cid * NPAD + sid * ROWS_PER_TILE, ROWS_PER_TILE)])

  return k(xa2, src2, dst2)


def _tc_finish(acc, x, Wl_lo, Wl_hi, b_l, W_r):
  BN = 1000

  def body(a0_ref, a1_ref, x_ref, wlo_ref, whi_ref, wr_ref, b_ref, o_ref):
    lo = a0_ref[0]
    hi = a1_ref[0]
    cnt = jnp.maximum(lo[:, DH:DH + 1], 1.0)
    mean_lo = lo[:, :DH] / cnt
    mean_hi = hi[:, :DH] / cnt
    dn = (((1,), (1,)), ((), ()))
    o_ref[...] = (
        lax.dot_general(mean_lo, wlo_ref[...], dn,
                        preferred_element_type=jnp.float32)
        + lax.dot_general(mean_hi, whi_ref[...], dn,
                          preferred_element_type=jnp.float32)
        + lax.dot_general(x_ref[...], wr_ref[...], dn,
                          preferred_element_type=jnp.float32)
        + b_ref[...])

  return pl.pallas_call(
      body,
      grid=(N // BN,),
      in_specs=[
          pl.BlockSpec((1, BN, DW),
                       lambda i: (jnp.int32(0), i, jnp.int32(0))),
          pl.BlockSpec((1, BN, DW),
                       lambda i: (jnp.int32(1), i, jnp.int32(0))),
          pl.BlockSpec((BN, D), lambda i: (i, jnp.int32(0))),
          pl.BlockSpec((D, DH), lambda i: (jnp.int32(0), jnp.int32(0))),
          pl.BlockSpec((D, DH), lambda i: (jnp.int32(0), jnp.int32(0))),
          pl.BlockSpec((D, D), lambda i: (jnp.int32(0), jnp.int32(0))),
          pl.BlockSpec((1, D), lambda i: (jnp.int32(0), jnp.int32(0))),
      ],
      out_specs=pl.BlockSpec((BN, D), lambda i: (i, jnp.int32(0))),
      out_shape=jax.ShapeDtypeStruct((N, D), jnp.float32),
  )(acc, acc, x, Wl_lo, Wl_hi, W_r, b_l.reshape(1, D))


def kernel(x, edge_index, edge_attr, W_l, b_l, W_r):
  src = edge_index[0].astype(jnp.int32)
  dst = edge_index[1].astype(jnp.int32)
  xf = x.astype(jnp.float32)

  xa2 = jnp.zeros((2 * NPAD, DW), jnp.float32)
  xa2 = xa2.at[:N, :DH].set(xf[:, :DH])
  xa2 = xa2.at[:N, DH].set(1.0)
  xa2 = xa2.at[NPAD:NPAD + N, :DH].set(xf[:, DH:])

  pad = EPAD - E
  src_p = jnp.concatenate([src, jnp.zeros((pad,), jnp.int32)])
  dst_p = jnp.concatenate([dst, jnp.full((pad,), NPAD - 1, jnp.int32)])
  src3 = src_p.reshape(16, NCHUNK, K)
  dst3 = dst_p.reshape(16, NCHUNK, K)

  acc = _sc_segment_sum(xa2, src3, dst3).reshape(2, NPAD, DW)
  Wl = W_l.astype(jnp.float32)
  out = _tc_finish(acc, xf, Wl[:, :DH], Wl[:, DH:],
                   b_l.astype(jnp.float32), W_r.astype(jnp.float32))
  out_dtype = jnp.result_type(x.dtype, W_l.dtype)
  return out.astype(out_dtype)


# speedup vs baseline: 1.2068x; 1.2003x over previous
import functools

import jax
import jax.numpy as jnp
from jax import lax
from jax.experimental import pallas as pl
from jax.experimental.pallas import tpu as pltpu
from jax.experimental.pallas import tpu_sc as plsc

N = 10000
E = 320000
D = 128
DH = 64
DW = 80
NPAD = 10240
K = 64
NCHUNKS = E // K
CPT = 320
IB = 5
NBUF = 5
ROWS_PER_TILE = NPAD // 16
XPT = N // 16


def _sc_segment_sum(x, src2d, dst2d):
  mesh = plsc.VectorSubcoreMesh(core_axis_name="c", subcore_axis_name="s")

  @functools.partial(
      pl.kernel,
      mesh=mesh,
      out_type=jax.ShapeDtypeStruct((2 * NPAD, DW), jnp.float32),
      scratch_types=[
          pltpu.VMEM((4, IB, K), jnp.int32),
          pltpu.VMEM((4, IB, K), jnp.int32),
          pltpu.VMEM((NBUF, K, DW), jnp.float32),
          pltpu.VMEM_SHARED((NPAD, DW), jnp.float32),
          pltpu.VMEM_SHARED((N, DW), jnp.float32),
          [pltpu.SemaphoreType.DMA] * NBUF,
          [pltpu.SemaphoreType.DMA] * NBUF,
          [pltpu.SemaphoreType.DMA] * 4,
          [pltpu.SemaphoreType.DMA] * 4,
      ],
      compiler_params=pltpu.CompilerParams(use_tc_tiling_on_sc=False),
  )
  def k(x_hbm, src_hbm, dst_hbm, stamp_hbm, out_hbm,
        src_v, dst_v, rows_v, acc_sh, xtab_sh, sem_g, sem_s,
        sem_is, sem_id):
    cid = lax.axis_index("c")
    sid = lax.axis_index("s")

    xr0 = sid * XPT
    pltpu.async_copy(x_hbm.at[pl.ds(xr0, XPT), pl.ds(cid * DH, DH)],
                     xtab_sh.at[pl.ds(xr0, XPT), pl.ds(0, DH)], sem_g[0])

    def zrow(r, carry):
      for c in range(DW // 16):
        rows_v[jnp.int32(0), r, pl.ds(c * 16, 16)] = jnp.zeros(
            (16,), jnp.float32)
      return carry

    lax.fori_loop(jnp.int32(0), jnp.int32(K), zrow, jnp.int32(0))
    pltpu.make_async_copy(x_hbm.at[pl.ds(0, XPT), pl.ds(0, DH)],
                          xtab_sh.at[pl.ds(0, XPT), pl.ds(0, DH)],
                          sem_g[0]).wait()

    def zslab(j, carry):
      pltpu.sync_copy(rows_v.at[jnp.int32(0)],
                      acc_sh.at[pl.ds(sid * ROWS_PER_TILE + j * K, K)])
      return carry

    lax.fori_loop(jnp.int32(0), jnp.int32(ROWS_PER_TILE // K), zslab,
                  jnp.int32(0))

    def ocol(q, carry):
      pltpu.sync_copy(stamp_hbm,
                      xtab_sh.at[pl.ds(xr0 + q * 125, 125), pl.ds(DH, 16)])
      return carry

    lax.fori_loop(jnp.int32(0), jnp.int32(XPT // 125), ocol, jnp.int32(0))
    plsc.subcore_barrier()

    cbase = sid * CPT
    nchunk = jnp.where(sid == 15, jnp.int32(NCHUNKS - 15 * CPT),
                       jnp.int32(CPT))
    nblk = nchunk // jnp.int32(IB)

    def load_idx_block(j, p):
      pi = jnp.int32(p)
      pltpu.async_copy(src_hbm.at[pl.ds(cbase + j * IB, IB)], src_v.at[pi],
                       sem_is[p])
      pltpu.async_copy(dst_hbm.at[pl.ds(cbase + j * IB, IB)], dst_v.at[pi],
                       sem_id[p])

    def wait_idx_block(p):
      pi = jnp.int32(p)
      pltpu.make_async_copy(src_hbm.at[pl.ds(0, IB)], src_v.at[pi],
                            sem_is[p]).wait()
      pltpu.make_async_copy(dst_hbm.at[pl.ds(0, IB)], dst_v.at[pi],
                            sem_id[p]).wait()

    load_idx_block(jnp.int32(0), 0)
    load_idx_block(jnp.int32(1), 1)
    load_idx_block(jnp.int32(2), 2)
    wait_idx_block(0)

    for b in range(NBUF):
      bi = jnp.int32(b)
      pltpu.async_copy(xtab_sh.at[src_v.at[jnp.int32(0), bi]],
                       rows_v.at[bi], sem_g[b])

    scat_dummy = out_hbm.at[pl.ds(0, K)]

    def block_quad(i, carry):
      for q in range(4):
        p, pn, pl3 = q, (q + 1) % 4, (q + 3) % 4
        j = i * jnp.int32(4) + jnp.int32(q)
        have_next = j + jnp.int32(1) < nblk

        @pl.when(have_next)
        def _():
          wait_idx_block(pn)

        for u in range(IB):
          bi = jnp.int32(u)
          pltpu.make_async_copy(scat_dummy, rows_v.at[bi],
                                sem_g[u]).wait()
          pltpu.async_copy(rows_v.at[bi],
                           acc_sh.at[dst_v.at[jnp.int32(p), bi]],
                           sem_s[u], add=True)

          @pl.when(have_next)
          def _():
            pltpu.make_async_copy(scat_dummy, rows_v.at[bi],
                                  sem_s[u]).wait()
            pltpu.async_copy(xtab_sh.at[src_v.at[jnp.int32(pn), bi]],
                             rows_v.at[bi], sem_g[u])

        @pl.when(j + jnp.int32(3) < nblk)
        def _():
          load_idx_block(j + jnp.int32(3), pl3)
      return carry

    lax.fori_loop(jnp.int32(0), nblk // jnp.int32(4), block_quad,
                  jnp.int32(0), unroll=False)

    for b in range(NBUF):
      bi = jnp.int32(b)
      pltpu.make_async_copy(scat_dummy, rows_v.at[bi], sem_s[b]).wait()

    plsc.subcore_barrier()

    pltpu.sync_copy(
        acc_sh.at[pl.ds(sid * ROWS_PER_TILE, ROWS_PER_TILE)],
        out_hbm.at[pl.ds(cid * NPAD + sid * ROWS_PER_TILE, ROWS_PER_TILE)])

  stamp = jnp.zeros((125, 16), jnp.float32).at[:, 0].set(1.0)
  return k(x, src2d, dst2d, stamp)


def _tc_finish(acc, x, Wl_lo, Wl_hi, b_l, W_r):
  BN = 640

  def body(a0_ref, a1_ref, x_ref, wlo_ref, whi_ref, wr_ref, b_ref, o_ref):
    lo = a0_ref[...]
    hi = a1_ref[...]
    cnt = jnp.maximum(lo[:, DH:DH + 1], 1.0)
    mean_lo = lo[:, :DH] / cnt
    mean_hi = hi[:, :DH] / cnt
    dn = (((1,), (1,)), ((), ()))
    o_ref[...] = (
        lax.dot_general(mean_lo, wlo_ref[...], dn,
                        preferred_element_type=jnp.float32)
        + lax.dot_general(mean_hi, whi_ref[...], dn,
                          preferred_element_type=jnp.float32)
        + lax.dot_general(x_ref[...], wr_ref[...], dn,
                          preferred_element_type=jnp.float32)
        + b_ref[...])

  nblk = NPAD // BN
  return pl.pallas_call(
      body,
      grid=(pl.cdiv(N, BN),),
      in_specs=[
          pl.BlockSpec((BN, DW), lambda i: (i, jnp.int32(0))),
          pl.BlockSpec((BN, DW), lambda i: (i + jnp.int32(nblk),
                                            jnp.int32(0))),
          pl.BlockSpec((BN, D), lambda i: (i, jnp.int32(0))),
          pl.BlockSpec((D, DH), lambda i: (jnp.int32(0), jnp.int32(0))),
          pl.BlockSpec((D, DH), lambda i: (jnp.int32(0), jnp.int32(0))),
          pl.BlockSpec((D, D), lambda i: (jnp.int32(0), jnp.int32(0))),
          pl.BlockSpec((1, D), lambda i: (jnp.int32(0), jnp.int32(0))),
      ],
      out_specs=pl.BlockSpec((BN, D), lambda i: (i, jnp.int32(0))),
      out_shape=jax.ShapeDtypeStruct((N, D), jnp.float32),
  )(acc, acc, x, Wl_lo, Wl_hi, W_r, b_l.reshape(1, D))


def kernel(x, edge_index, edge_attr, W_l, b_l, W_r):
  src2d = edge_index[0].astype(jnp.int32).reshape(NCHUNKS, K)
  dst2d = edge_index[1].astype(jnp.int32).reshape(NCHUNKS, K)
  xf = x.astype(jnp.float32)

  acc = _sc_segment_sum(xf, src2d, dst2d)
  Wl = W_l.astype(jnp.float32)
  out = _tc_finish(acc, xf, Wl[:, :DH], Wl[:, DH:],
                   b_l.astype(jnp.float32), W_r.astype(jnp.float32))
  out_dtype = jnp.result_type(x.dtype, W_l.dtype)
  return out.astype(out_dtype)
